# baseline (speedup 1.0000x reference)
import functools

import jax
import jax.numpy as jnp
from jax import lax
from jax.experimental import pallas as pl
from jax.experimental.pallas import tpu as pltpu
from jax.experimental.pallas import tpu_sc as plsc

_B = 4096
_T = 200
_EMB = 32
_N = _B * _T
_SPLIT = 8
_NH = _N // _SPLIT

_NC = 2
_NS = 16
_NW = _NC * _NS
_PER_W = _NH // _NW
_CHUNK = 1600
_NCHUNK = _PER_W // _CHUNK

_mesh = plsc.VectorSubcoreMesh(core_axis_name="c", subcore_axis_name="s")


@functools.partial(
    pl.kernel,
    mesh=_mesh,
    out_type=jax.ShapeDtypeStruct((_NH, _EMB), jnp.float32),
    scratch_types=[
        pltpu.VMEM((_NCHUNK, _CHUNK), jnp.int32),
        pltpu.VMEM((_CHUNK, _EMB), jnp.float32),
        pltpu.VMEM((_CHUNK, _EMB), jnp.float32),
        pltpu.SemaphoreType.DMA,
        pltpu.SemaphoreType.DMA,
        pltpu.SemaphoreType.DMA,
        pltpu.SemaphoreType.DMA,
    ],
    compiler_params=pltpu.CompilerParams(use_tc_tiling_on_sc=False),
)
def _sc_gather(idx_hbm, table_hbm, out_hbm, idx_v, rows0, rows1,
               sg0, sg1, sw0, sw1):
    wid = lax.axis_index("s") * _NC + lax.axis_index("c")
    base = wid * _PER_W
    rows = (rows0, rows1)
    sg = (sg0, sg1)
    sw = (sw0, sw1)

    pltpu.sync_copy(idx_hbm.at[wid], idx_v)

    @pl.loop(0, _NCHUNK // 2)
    def body(g):
        for b in range(2):
            i = 2 * g + b

            @pl.when(g > 0)
            def _wait_wb():
                pltpu.make_async_copy(
                    rows[b], out_hbm.at[pl.ds(base, _CHUNK)], sw[b]).wait()

            pltpu.async_copy(table_hbm.at[idx_v.at[i]], rows[b], sg[b])

        for b in range(2):
            i = 2 * g + b
            pltpu.make_async_copy(
                table_hbm.at[idx_v.at[i]], rows[b], sg[b]).wait()
            pltpu.async_copy(
                rows[b], out_hbm.at[pl.ds(base + i * _CHUNK, _CHUNK)], sw[b])

    for b in range(2):
        pltpu.make_async_copy(
            rows[b], out_hbm.at[pl.ds(base, _CHUNK)], sw[b]).wait()


def kernel(x, table):
    xf = x.reshape(_N)
    halves = []
    for s in range(_SPLIT):
        xs = xf[s * _NH:(s + 1) * _NH]
        flat = _sc_gather(xs.reshape(_NW, _NCHUNK, _CHUNK), table)
        halves.append(flat.reshape(_B // _SPLIT, _T, _EMB))
    return jnp.concatenate(halves, axis=0)


# speedup vs baseline: 1.0155x; 1.0155x over previous
import functools

import jax
import jax.numpy as jnp
from jax import lax
from jax.experimental import pallas as pl
from jax.experimental.pallas import tpu as pltpu
from jax.experimental.pallas import tpu_sc as plsc

_B = 4096
_T = 200
_EMB = 32
_N = _B * _T
_SPLIT = 4
_NH = _N // _SPLIT

_NC = 2
_NS = 16
_NW = _NC * _NS
_PER_W = _NH // _NW
_CHUNK = 1600
_NCHUNK = _PER_W // _CHUNK

_mesh = plsc.VectorSubcoreMesh(core_axis_name="c", subcore_axis_name="s")


@functools.partial(
    pl.kernel,
    mesh=_mesh,
    out_type=jax.ShapeDtypeStruct((_NH, _EMB), jnp.float32),
    scratch_types=[
        pltpu.VMEM((_NCHUNK, _CHUNK), jnp.int32),
        pltpu.VMEM((_CHUNK, _EMB), jnp.float32),
        pltpu.VMEM((_CHUNK, _EMB), jnp.float32),
        pltpu.SemaphoreType.DMA,
        pltpu.SemaphoreType.DMA,
        pltpu.SemaphoreType.DMA,
        pltpu.SemaphoreType.DMA,
    ],
    compiler_params=pltpu.CompilerParams(use_tc_tiling_on_sc=False),
)
def _sc_gather(idx_hbm, table_hbm, out_hbm, idx_v, rows0, rows1,
               sg0, sg1, sw0, sw1):
    wid = lax.axis_index("s") * _NC + lax.axis_index("c")
    base = wid * _PER_W
    rows = (rows0, rows1)
    sg = (sg0, sg1)
    sw = (sw0, sw1)

    pltpu.sync_copy(idx_hbm.at[wid], idx_v)

    @pl.loop(0, _NCHUNK // 2)
    def body(g):
        for b in range(2):
            i = 2 * g + b

            @pl.when(g > 0)
            def _wait_wb():
                pltpu.make_async_copy(
                    rows[b], out_hbm.at[pl.ds(base, _CHUNK)], sw[b]).wait()

            pltpu.async_copy(table_hbm.at[idx_v.at[i]], rows[b], sg[b])

        for b in range(2):
            i = 2 * g + b
            pltpu.make_async_copy(
                table_hbm.at[idx_v.at[i]], rows[b], sg[b]).wait()
            pltpu.async_copy(
                rows[b], out_hbm.at[pl.ds(base + i * _CHUNK, _CHUNK)], sw[b])

    for b in range(2):
        pltpu.make_async_copy(
            rows[b], out_hbm.at[pl.ds(base, _CHUNK)], sw[b]).wait()


def kernel(x, table):
    xf = x.reshape(_N)
    halves = []
    for s in range(_SPLIT):
        xs = xf[s * _NH:(s + 1) * _NH]
        flat = _sc_gather(xs.reshape(_NW, _NCHUNK, _CHUNK), table)
        halves.append(flat.reshape(_B // _SPLIT, _T, _EMB))
    return jnp.concatenate(halves, axis=0)
